# Initial kernel scaffold; baseline (speedup 1.0000x reference)
import jax
import jax.numpy as jnp
from jax.experimental import pallas as pl


def kernel(x, edge_index, W1, b1, Wc, bc, c_conv, c_dec):
    raise NotImplementedError("write your pallas kernel here")


# speedup vs baseline: 13.4246x; 13.4246x over previous
import functools

import jax
import jax.numpy as jnp
from jax import lax
from jax.experimental import pallas as pl
from jax.experimental.pallas import tpu as pltpu
from jax.experimental.pallas import tpu_sc as plsc

_NC = 2
_NS = 16


def _artanh(z):
    z = jnp.clip(z, -1.0 + 1e-7, 1.0 - 1e-7)
    return 0.5 * (jnp.log1p(z) - jnp.log1p(-z))


def _rownorm(x):
    return jnp.clip(jnp.sqrt(jnp.sum(x * x, axis=-1, keepdims=True)), 1e-15, None)


def _sc_deg_body(n_pad, k_tile, ch, dst_hbm, zeros_hbm, out_hbm,
                 idx_v, ones_v, deg_sh, sem):
    c = lax.axis_index("c")
    s = lax.axis_index("s")

    for i in range(ch // 16):
        ones_v[pl.ds(i * 16, 16)] = jnp.ones((16,), jnp.float32)

    @pl.when(s == 0)
    def _():
        pltpu.sync_copy(zeros_hbm, deg_sh)

    plsc.subcore_barrier()

    base = (c * _NS + s) * k_tile

    def body(j, carry):
        off = pl.multiple_of(base + j * ch, 8)
        pltpu.sync_copy(dst_hbm.at[pl.ds(off, ch)], idx_v)
        pltpu.sync_copy(ones_v, deg_sh.at[idx_v], add=True)
        return carry

    lax.fori_loop(0, k_tile // ch, body, 0)

    plsc.subcore_barrier()

    @pl.when(s == 0)
    def _():
        pltpu.sync_copy(deg_sh, out_hbm.at[pl.ds(c * n_pad, n_pad)])


def _sc_agg_body(n_pad, d, k_tile, ch, hp_hbm, src_hbm, dst_hbm, zeros_hbm,
                 out_hbm, sidx_v, didx_v, rows_v, acc_sh, sem):
    c = lax.axis_index("c")
    s = lax.axis_index("s")
    rows_per_tile = n_pad // _NS

    pltpu.sync_copy(zeros_hbm.at[pl.ds(s * rows_per_tile, rows_per_tile)],
                    acc_sh.at[pl.ds(s * rows_per_tile, rows_per_tile)])
    plsc.subcore_barrier()

    base = (c * _NS + s) * k_tile

    def body(j, carry):
        off = pl.multiple_of(base + j * ch, 8)
        pltpu.sync_copy(src_hbm.at[pl.ds(off, ch)], sidx_v)
        pltpu.sync_copy(dst_hbm.at[pl.ds(off, ch)], didx_v)
        pltpu.async_copy(hp_hbm.at[sidx_v], rows_v, sem).wait()
        pltpu.sync_copy(rows_v, acc_sh.at[didx_v], add=True)
        return carry

    lax.fori_loop(0, k_tile // ch, body, 0)

    plsc.subcore_barrier()

    pltpu.sync_copy(acc_sh.at[pl.ds(s * rows_per_tile, rows_per_tile)],
                    out_hbm.at[c, pl.ds(s * rows_per_tile, rows_per_tile)])


def _tc1_body(x_ref, w1_ref, b1_ref, deg_ref, c1_ref, out_ref):
    x = x_ref[...]
    c1 = c1_ref[...][0]
    sc = jnp.sqrt(c1)

    n = _rownorm(x)
    maxn = (1.0 - 1e-5) / sc
    n_clip = jnp.minimum(n, maxn)
    t = x * (_artanh(sc * n_clip) / (sc * n))

    h = jnp.dot(t, w1_ref[...], preferred_element_type=jnp.float32)
    h = h + b1_ref[...][None, :]

    d2 = deg_ref[...]
    n_pad = d2.shape[0] // 2
    deg = jnp.maximum(d2[:n_pad] + d2[n_pad:], 1.0)
    u = lax.rsqrt(deg)
    out_ref[...] = h * u[: h.shape[0], None]


def _tc2_body(raw_ref, deg_ref, wc_ref, bc_ref, c1_ref, c2_ref, out_ref):
    r = raw_ref[...]
    d2 = deg_ref[...]
    n_pad = d2.shape[0] // 2
    deg = jnp.maximum(d2[:n_pad] + d2[n_pad:], 1.0)
    u = lax.rsqrt(deg)

    agg = (r[0] + r[1]) * u[:, None]
    h = jnp.maximum(agg, 0.0)

    c1 = c1_ref[...][0]
    c2 = c2_ref[...][0]
    sc1 = jnp.sqrt(c1)
    sc2 = jnp.sqrt(c2)

    m = _rownorm(h)
    y = jnp.tanh(sc1 * m) * h / (sc1 * m)
    yn = _rownorm(y)
    maxn = (1.0 - 1e-5) / sc1
    y = jnp.where(yn > maxn, y / yn * maxn, y)
    n2 = _rownorm(y)
    g = y * (_artanh(sc2 * n2) / (sc2 * n2))

    logits = jnp.dot(g, wc_ref[...], preferred_element_type=jnp.float32)
    logits = logits + bc_ref[...][None, :]

    mx = jnp.max(logits, axis=-1, keepdims=True)
    sh = logits - mx
    lse = jnp.log(jnp.sum(jnp.exp(sh), axis=-1, keepdims=True))
    out_ref[...] = sh - lse


def _pick_chunk(k_tile):
    for ch in range(128, 7, -8):
        if k_tile % ch == 0:
            return ch
    return 8


def kernel(x, edge_index, W1, b1, Wc, bc, c_conv, c_dec):
    n_nodes, d_in = x.shape
    d_hid = W1.shape[1]
    d_out = Wc.shape[1]
    e = edge_index.shape[1]

    src = edge_index[0]
    dst = edge_index[1]

    k_tile = e // (_NC * _NS)
    ch = _pick_chunk(k_tile)

    n_pad = ((n_nodes + 127) // 128) * 128

    zeros_vec = jnp.zeros((n_pad,), jnp.float32)
    zeros_mat = jnp.zeros((n_pad, d_hid), jnp.float32)

    mesh = plsc.VectorSubcoreMesh(core_axis_name="c", subcore_axis_name="s")

    deg_fn = pl.kernel(
        functools.partial(_sc_deg_body, n_pad, k_tile, ch),
        out_type=jax.ShapeDtypeStruct((_NC * n_pad,), jnp.float32),
        mesh=mesh,
        scratch_types=[
            pltpu.VMEM((ch,), jnp.int32),
            pltpu.VMEM((ch,), jnp.float32),
            pltpu.VMEM_SHARED((n_pad,), jnp.float32),
            pltpu.SemaphoreType.DMA,
        ],
    )
    deg = deg_fn(dst, zeros_vec)

    hp = pl.pallas_call(
        _tc1_body,
        out_shape=jax.ShapeDtypeStruct((n_nodes, d_hid), jnp.float32),
    )(x, W1, b1, deg, c_conv)

    agg_fn = pl.kernel(
        functools.partial(_sc_agg_body, n_pad, d_hid, k_tile, ch),
        out_type=jax.ShapeDtypeStruct((_NC, n_pad, d_hid), jnp.float32),
        mesh=mesh,
        scratch_types=[
            pltpu.VMEM((ch,), jnp.int32),
            pltpu.VMEM((ch,), jnp.int32),
            pltpu.VMEM((ch, d_hid), jnp.float32),
            pltpu.VMEM_SHARED((n_pad, d_hid), jnp.float32),
            pltpu.SemaphoreType.DMA,
        ],
    )
    raw = agg_fn(hp, src, dst, zeros_mat)

    out = pl.pallas_call(
        _tc2_body,
        out_shape=jax.ShapeDtypeStruct((n_pad, d_out), jnp.float32),
    )(raw, deg, Wc, bc, c_conv, c_dec)

    return out[:n_nodes]
